# baseline (speedup 1.0000x reference)
import functools

import jax
import jax.numpy as jnp
from jax import lax
from jax.experimental import pallas as pl
from jax.experimental.pallas import tpu as pltpu
from jax.experimental.pallas import tpu_sc as plsc

_B = 64
_CAND = 32
_SW = 64
_C = 100
_K = 10
_NC = 2
_NS = 16
_NW = _NC * _NS
_SLABS = _B * _CAND // _NW
_G = 16
_GPS = _SW // _G


def _insert(ms, x):
    out = [jnp.maximum(x, ms[0])]
    for i in range(1, len(ms)):
        out.append(jnp.maximum(jnp.minimum(x, ms[i - 1]), ms[i]))
    return tuple(out)


@functools.partial(
    pl.kernel,
    out_type=jax.ShapeDtypeStruct((_B, _CAND, _SW, _C), jnp.float32),
    mesh=plsc.VectorSubcoreMesh(core_axis_name="c", subcore_axis_name="s"),
    compiler_params=pltpu.CompilerParams(needs_layout_passes=False),
    scratch_types=[
        pltpu.VMEM((_SW, _C), jnp.float32),
        pltpu.VMEM((_SW, _C), jnp.float32),
        pltpu.VMEM((_G * _C,), jnp.float32),
        pltpu.VMEM((2, _C), jnp.float32),
    ],
)
def _topk_mask_norm(a_hbm, mask_hbm, out_hbm, inbuf, outbuf, tbuf, mask_v):
    wid = lax.axis_index("s") * _NC + lax.axis_index("c")
    pltpu.sync_copy(mask_hbm.at[pl.ds(2 * wid, 2)], mask_v)
    iota = lax.iota(jnp.int32, 16)
    neg_inf = jnp.full((16,), -jnp.inf, jnp.float32)
    zero = jnp.zeros((16,), jnp.float32)

    def slab_body(ci, carry):
        b = 2 * wid + ci // _CAND
        c = ci % _CAND
        b_local = ci // _CAND
        pltpu.sync_copy(a_hbm.at[b, c], inbuf)

        def group_body(g, carry2):
            rows = g * _G + iota

            @plsc.parallel_loop(0, _C, carry=(neg_inf,) * _K, unroll=4)
            def p1(j, ms):
                x = plsc.load_gather(inbuf, [rows, jnp.broadcast_to(j, (_G,))])
                tbuf[pl.ds(j * _G, _G)] = x
                return _insert(ms, x)

            ms = p1
            m0, thr = ms[0], ms[_K - 1]

            @plsc.parallel_loop(0, _C, carry=(zero, zero), unroll=4)
            def p2(j, zs):
                z, s = zs
                x = tbuf[pl.ds(j * _G, _G)]
                e = jnp.where(x >= thr, jnp.exp(x - m0), 0.0)
                mvec = plsc.load_gather(
                    mask_v,
                    [jnp.broadcast_to(b_local, (_G,)).astype(jnp.int32),
                     jnp.broadcast_to(j, (_G,)).astype(jnp.int32)],
                )
                t = e * mvec
                tbuf[pl.ds(j * _G, _G)] = t
                return (z + e, s + t)

            z, s = p2
            inv = 1.0 / (s + 1e-8 * z)

            @plsc.parallel_loop(0, _C, unroll=4)
            def p3(j):
                t = tbuf[pl.ds(j * _G, _G)]
                plsc.store_scatter(
                    outbuf, [rows, jnp.broadcast_to(j, (_G,))], t * inv)

            return carry2

        lax.fori_loop(0, _GPS, group_body, 0)
        pltpu.sync_copy(outbuf, out_hbm.at[b, c])
        return carry

    lax.fori_loop(0, _SLABS, slab_body, 0)


def kernel(a, mask):
    return _topk_mask_norm(a, mask)


# speedup vs baseline: 1.0444x; 1.0444x over previous
import functools

import jax
import jax.numpy as jnp
from jax import lax
from jax.experimental import pallas as pl
from jax.experimental.pallas import tpu as pltpu
from jax.experimental.pallas import tpu_sc as plsc

_B = 64
_CAND = 32
_SW = 64
_C = 100
_K = 10
_R = _B * _CAND * _SW
_NC = 2
_NS = 16
_NW = _NC * _NS
_RPW = _R // _NW
_CHUNK = 128
_NCHUNK = _RPW // _CHUNK
_G = 16
_GPC = _CHUNK // _G
_RPB = _CAND * _SW
_CPB = _RPB // _CHUNK


def _insert(ms, x):
    out = [jnp.maximum(x, ms[0])]
    for i in range(1, len(ms)):
        out.append(jnp.maximum(jnp.minimum(x, ms[i - 1]), ms[i]))
    return tuple(out)


@functools.partial(
    pl.kernel,
    out_type=jax.ShapeDtypeStruct((_R, _C), jnp.float32),
    mesh=plsc.VectorSubcoreMesh(core_axis_name="c", subcore_axis_name="s"),
    compiler_params=pltpu.CompilerParams(needs_layout_passes=False),
    scratch_types=[
        pltpu.VMEM((_CHUNK, _C), jnp.float32),
        pltpu.VMEM((_CHUNK, _C), jnp.float32),
        pltpu.VMEM((_G * _C,), jnp.float32),
        pltpu.VMEM((2, _C), jnp.float32),
    ],
)
def _topk_mask_norm(a_hbm, mask_hbm, out_hbm, inbuf, outbuf, tbuf, mask_v):
    wid = lax.axis_index("s") * _NC + lax.axis_index("c")
    row0 = wid * _RPW
    pltpu.sync_copy(mask_hbm.at[pl.ds(2 * wid, 2)], mask_v)
    iota = lax.iota(jnp.int32, 16)
    neg_inf = jnp.full((16,), -jnp.inf, jnp.float32)
    zero = jnp.zeros((16,), jnp.float32)

    def chunk_body(ci, carry):
        crow = row0 + ci * _CHUNK
        pltpu.sync_copy(a_hbm.at[pl.ds(crow, _CHUNK)], inbuf)
        b_local = ci // _CPB

        def group_body(g, carry2):
            rows = g * _G + iota

            @plsc.parallel_loop(0, _C, carry=(neg_inf,) * _K, unroll=4)
            def p1(j, ms):
                x = plsc.load_gather(inbuf, [rows, jnp.broadcast_to(j, (_G,))])
                tbuf[pl.ds(j * _G, _G)] = x
                return _insert(ms, x)

            ms = p1
            m0, thr = ms[0], ms[_K - 1]

            @plsc.parallel_loop(0, _C, carry=(zero, zero), unroll=4)
            def p2(j, zs):
                z, s = zs
                x = tbuf[pl.ds(j * _G, _G)]
                e = jnp.where(x >= thr, jnp.exp(x - m0), 0.0)
                mvec = plsc.load_gather(
                    mask_v,
                    [jnp.broadcast_to(b_local, (_G,)).astype(jnp.int32),
                     jnp.broadcast_to(j, (_G,)).astype(jnp.int32)],
                )
                t = e * mvec
                tbuf[pl.ds(j * _G, _G)] = t
                return (z + e, s + t)

            z, s = p2
            inv = 1.0 / (s + 1e-8 * z)

            @plsc.parallel_loop(0, _C, unroll=4)
            def p3(j):
                t = tbuf[pl.ds(j * _G, _G)]
                plsc.store_scatter(
                    outbuf, [rows, jnp.broadcast_to(j, (_G,))], t * inv)

            return carry2

        lax.fori_loop(0, _GPC, group_body, 0)
        pltpu.sync_copy(outbuf, out_hbm.at[pl.ds(crow, _CHUNK)])
        return carry

    lax.fori_loop(0, _NCHUNK, chunk_body, 0)


def kernel(a, mask):
    out = _topk_mask_norm(a.reshape(_R, _C), mask)
    return out.reshape(_B, _CAND, _SW, _C)


# speedup vs baseline: 1.1640x; 1.1145x over previous
import functools

import jax
import jax.numpy as jnp
from jax import lax
from jax.experimental import pallas as pl
from jax.experimental.pallas import tpu as pltpu
from jax.experimental.pallas import tpu_sc as plsc

_B = 64
_CAND = 32
_SW = 64
_C = 100
_K = 10
_R = _B * _CAND * _SW
_NC = 2
_NS = 16
_NW = _NC * _NS
_RPW = _R // _NW
_CHUNK = 128
_NCHUNK = _RPW // _CHUNK
_G = 16
_GPC = _CHUNK // _G
_RPB = _CAND * _SW
_CPB = _RPB // _CHUNK


def _insert(ms, x):
    out = [jnp.maximum(x, ms[0])]
    for i in range(1, len(ms)):
        out.append(jnp.maximum(jnp.minimum(x, ms[i - 1]), ms[i]))
    return tuple(out)


@functools.partial(
    pl.kernel,
    out_type=jax.ShapeDtypeStruct((_R, _C), jnp.float32),
    mesh=plsc.VectorSubcoreMesh(core_axis_name="c", subcore_axis_name="s"),
    compiler_params=pltpu.CompilerParams(needs_layout_passes=False),
    scratch_types=[
        pltpu.VMEM((_CHUNK, _C), jnp.float32),
        pltpu.VMEM((_CHUNK, _C), jnp.float32),
        pltpu.VMEM((_CHUNK, _C), jnp.float32),
        pltpu.VMEM((_CHUNK, _C), jnp.float32),
        pltpu.VMEM((_G * _C,), jnp.float32),
        pltpu.VMEM((2, _C), jnp.float32),
        pltpu.SemaphoreType.DMA,
        pltpu.SemaphoreType.DMA,
        pltpu.SemaphoreType.DMA,
        pltpu.SemaphoreType.DMA,
    ],
)
def _topk_mask_norm(a_hbm, mask_hbm, out_hbm, in0, in1, out0, out1, tbuf,
                    mask_v, sin0, sin1, sout0, sout1):
    wid = lax.axis_index("s") * _NC + lax.axis_index("c")
    row0 = wid * _RPW
    pltpu.sync_copy(mask_hbm.at[pl.ds(2 * wid, 2)], mask_v)
    iota = lax.iota(jnp.int32, 16)
    neg_inf = jnp.full((16,), -jnp.inf, jnp.float32)
    zero = jnp.zeros((16,), jnp.float32)

    def in_copy(ci, buf, sem):
        crow = row0 + ci * _CHUNK
        return pltpu.make_async_copy(a_hbm.at[pl.ds(crow, _CHUNK)], buf, sem)

    def out_copy(ci, buf, sem):
        crow = row0 + ci * _CHUNK
        return pltpu.make_async_copy(buf, out_hbm.at[pl.ds(crow, _CHUNK)], sem)

    def compute(ci, inbuf, outbuf):
        b_local = ci // _CPB

        def group_body(g, carry2):
            rows = g * _G + iota

            @plsc.parallel_loop(0, _C, carry=(neg_inf,) * _K, unroll=4)
            def p1(j, ms):
                x = plsc.load_gather(inbuf, [rows, jnp.broadcast_to(j, (_G,))])
                tbuf[pl.ds(j * _G, _G)] = x
                return _insert(ms, x)

            ms = p1
            m0, thr = ms[0], ms[_K - 1]

            @plsc.parallel_loop(0, _C, carry=(zero, zero), unroll=4)
            def p2(j, zs):
                z, s = zs
                x = tbuf[pl.ds(j * _G, _G)]
                e = jnp.where(x >= thr, jnp.exp(x - m0), 0.0)
                mvec = plsc.load_gather(
                    mask_v,
                    [jnp.broadcast_to(b_local, (_G,)).astype(jnp.int32),
                     jnp.broadcast_to(j, (_G,)).astype(jnp.int32)],
                )
                t = e * mvec
                tbuf[pl.ds(j * _G, _G)] = t
                return (z + e, s + t)

            z, s = p2
            inv = 1.0 / (s + 1e-8 * z)

            @plsc.parallel_loop(0, _C, unroll=4)
            def p3(j):
                t = tbuf[pl.ds(j * _G, _G)]
                plsc.store_scatter(
                    outbuf, [rows, jnp.broadcast_to(j, (_G,))], t * inv)

            return carry2

        lax.fori_loop(0, _GPC, group_body, 0)

    in_copy(0, in0, sin0).start()

    def pair_body(k, carry):
        ci0 = 2 * k
        ci1 = ci0 + 1
        in_copy(ci0, in0, sin0).wait()
        in_copy(ci1, in1, sin1).start()

        @pl.when(k > 0)
        def _():
            out_copy(ci0 - 2, out0, sout0).wait()

        compute(ci0, in0, out0)
        out_copy(ci0, out0, sout0).start()

        in_copy(ci1, in1, sin1).wait()

        @pl.when(k < _NCHUNK // 2 - 1)
        def _():
            in_copy(ci1 + 1, in0, sin0).start()

        @pl.when(k > 0)
        def _():
            out_copy(ci1 - 2, out1, sout1).wait()

        compute(ci1, in1, out1)
        out_copy(ci1, out1, sout1).start()
        return carry

    lax.fori_loop(0, _NCHUNK // 2, pair_body, 0)
    out_copy(_NCHUNK - 2, out0, sout0).wait()
    out_copy(_NCHUNK - 1, out1, sout1).wait()


def kernel(a, mask):
    out = _topk_mask_norm(a.reshape(_R, _C), mask)
    return out.reshape(_B, _CAND, _SW, _C)


# speedup vs baseline: 1.1863x; 1.0192x over previous
import functools

import jax
import jax.numpy as jnp
from jax import lax
from jax.experimental import pallas as pl
from jax.experimental.pallas import tpu as pltpu
from jax.experimental.pallas import tpu_sc as plsc

_B = 64
_CAND = 32
_SW = 64
_C = 100
_K = 10
_R = _B * _CAND * _SW
_NC = 2
_NS = 16
_NW = _NC * _NS
_RPW = _R // _NW
_CHUNK = 128
_NCHUNK = _RPW // _CHUNK
_G = 16
_GPC = _CHUNK // _G
_RPB = _CAND * _SW
_CPB = _RPB // _CHUNK


def _insert(ms, x):
    out = [jnp.maximum(x, ms[0])]
    for i in range(1, len(ms)):
        out.append(jnp.maximum(jnp.minimum(x, ms[i - 1]), ms[i]))
    return tuple(out)


@functools.partial(
    pl.kernel,
    out_type=jax.ShapeDtypeStruct((_R, _C), jnp.float32),
    mesh=plsc.VectorSubcoreMesh(core_axis_name="c", subcore_axis_name="s"),
    compiler_params=pltpu.CompilerParams(needs_layout_passes=False),
    scratch_types=[
        pltpu.VMEM((_CHUNK, _C), jnp.float32),
        pltpu.VMEM((_CHUNK, _C), jnp.float32),
        pltpu.VMEM((_CHUNK, _C), jnp.float32),
        pltpu.VMEM((_CHUNK, _C), jnp.float32),
        pltpu.VMEM((_G * _C,), jnp.float32),
        pltpu.VMEM((2, _C), jnp.float32),
        pltpu.SemaphoreType.DMA,
        pltpu.SemaphoreType.DMA,
        pltpu.SemaphoreType.DMA,
        pltpu.SemaphoreType.DMA,
    ],
)
def _topk_mask_norm(a_hbm, mask_hbm, out_hbm, in0, in1, out0, out1, tbuf,
                    mask_v, sin0, sin1, sout0, sout1):
    wid = lax.axis_index("s") * _NC + lax.axis_index("c")
    row0 = wid * _RPW
    pltpu.sync_copy(mask_hbm.at[pl.ds(2 * wid, 2)], mask_v)
    iota = lax.iota(jnp.int32, 16)
    neg_inf = jnp.full((16,), -jnp.inf, jnp.float32)
    zero = jnp.zeros((16,), jnp.float32)

    def in_copy(ci, buf, sem):
        crow = row0 + ci * _CHUNK
        return pltpu.make_async_copy(a_hbm.at[pl.ds(crow, _CHUNK)], buf, sem)

    def out_copy(ci, buf, sem):
        crow = row0 + ci * _CHUNK
        return pltpu.make_async_copy(buf, out_hbm.at[pl.ds(crow, _CHUNK)], sem)

    def compute(ci, inbuf, outbuf):
        b_local = ci // _CPB

        def group_body(g, carry2):
            rows = g * _G + iota

            @plsc.parallel_loop(0, _C, carry=(neg_inf,) * _K, unroll=4)
            def p1(j, ms):
                jc = jnp.bitwise_and(j, 127)
                x = plsc.load_gather(inbuf, [rows, jnp.broadcast_to(jc, (_G,))])
                tbuf[pl.ds(j * _G, _G)] = x
                return _insert(ms, x)

            ms = p1
            m0, thr = ms[0], ms[_K - 1]

            @plsc.parallel_loop(0, _C, carry=(zero, zero), unroll=4)
            def p2(j, zs):
                z, s = zs
                x = tbuf[pl.ds(j * _G, _G)]
                e = jnp.where(x >= thr, jnp.exp(x - m0), 0.0)
                mvec = plsc.load_gather(
                    mask_v,
                    [jnp.broadcast_to(jnp.bitwise_and(b_local, 7), (_G,)),
                     jnp.broadcast_to(jnp.bitwise_and(j, 127), (_G,))],
                )
                t = e * mvec
                tbuf[pl.ds(j * _G, _G)] = t
                return (z + e, s + t)

            z, s = p2
            inv = 1.0 / (s + 1e-8 * z)

            @plsc.parallel_loop(0, _C, unroll=4)
            def p3(j):
                t = tbuf[pl.ds(j * _G, _G)]
                jc = jnp.bitwise_and(j, 127)
                plsc.store_scatter(
                    outbuf, [rows, jnp.broadcast_to(jc, (_G,))], t * inv)

            return carry2

        lax.fori_loop(0, _GPC, group_body, 0)

    in_copy(0, in0, sin0).start()

    def pair_body(k, carry):
        ci0 = 2 * k
        ci1 = ci0 + 1
        in_copy(ci0, in0, sin0).wait()
        in_copy(ci1, in1, sin1).start()

        @pl.when(k > 0)
        def _():
            out_copy(ci0 - 2, out0, sout0).wait()

        compute(ci0, in0, out0)
        out_copy(ci0, out0, sout0).start()

        in_copy(ci1, in1, sin1).wait()

        @pl.when(k < _NCHUNK // 2 - 1)
        def _():
            in_copy(ci1 + 1, in0, sin0).start()

        @pl.when(k > 0)
        def _():
            out_copy(ci1 - 2, out1, sout1).wait()

        compute(ci1, in1, out1)
        out_copy(ci1, out1, sout1).start()
        return carry

    lax.fori_loop(0, _NCHUNK // 2, pair_body, 0)
    out_copy(_NCHUNK - 2, out0, sout0).wait()
    out_copy(_NCHUNK - 1, out1, sout1).wait()


def kernel(a, mask):
    out = _topk_mask_norm(a.reshape(_R, _C), mask)
    return out.reshape(_B, _CAND, _SW, _C)


# speedup vs baseline: 2.4649x; 2.0777x over previous
import functools

import jax
import jax.numpy as jnp
from jax import lax
from jax.experimental import pallas as pl
from jax.experimental.pallas import tpu as pltpu
from jax.experimental.pallas import tpu_sc as plsc

_B = 64
_CAND = 32
_SW = 64
_C = 100
_K = 10
_R = _B * _CAND * _SW
_NC = 2
_NS = 16
_NW = _NC * _NS
_RPW = _R // _NW
_CHUNK = 128
_NCHUNK = _RPW // _CHUNK
_G = 16
_GPC = _CHUNK // _G
_RPB = _CAND * _SW
_CPB = _RPB // _CHUNK


def _insert(ms, x):
    out = [jnp.maximum(x, ms[0])]
    for i in range(1, len(ms)):
        out.append(jnp.maximum(jnp.minimum(x, ms[i - 1]), ms[i]))
    return tuple(out)


@functools.partial(
    pl.kernel,
    out_type=jax.ShapeDtypeStruct((_R, _C), jnp.float32),
    mesh=plsc.VectorSubcoreMesh(core_axis_name="c", subcore_axis_name="s"),
    compiler_params=pltpu.CompilerParams(needs_layout_passes=False),
    scratch_types=[
        pltpu.VMEM((_CHUNK, _C), jnp.float32),
        pltpu.VMEM((_CHUNK, _C), jnp.float32),
        pltpu.VMEM((_CHUNK, _C), jnp.float32),
        pltpu.VMEM((_CHUNK, _C), jnp.float32),
        pltpu.VMEM((_G * _C,), jnp.float32),
        pltpu.VMEM((2, _C), jnp.float32),
        pltpu.SemaphoreType.DMA,
        pltpu.SemaphoreType.DMA,
        pltpu.SemaphoreType.DMA,
        pltpu.SemaphoreType.DMA,
    ],
)
def _topk_mask_norm(a_hbm, mask_hbm, out_hbm, in0, in1, out0, out1, tbuf,
                    mask_v, sin0, sin1, sout0, sout1):
    wid = lax.axis_index("s") * _NC + lax.axis_index("c")
    row0 = wid * _RPW
    pltpu.sync_copy(mask_hbm.at[pl.ds(2 * wid, 2)], mask_v)
    iota = lax.iota(jnp.int32, 16)
    neg_inf = jnp.full((16,), -jnp.inf, jnp.float32)
    zero = jnp.zeros((16,), jnp.float32)

    def in_copy(ci, buf, sem):
        crow = row0 + ci * _CHUNK
        return pltpu.make_async_copy(a_hbm.at[pl.ds(crow, _CHUNK)], buf, sem)

    def out_copy(ci, buf, sem):
        crow = row0 + ci * _CHUNK
        return pltpu.make_async_copy(buf, out_hbm.at[pl.ds(crow, _CHUNK)], sem)

    def compute(ci, inbuf, outbuf):
        b_local = ci // _CPB
        b_vec = jnp.broadcast_to(jnp.bitwise_and(b_local, 7), (_G,))

        def group_body(g, carry2):
            rows = g * _G + iota

            def cols_of(t):
                return jnp.bitwise_and(t, -16) + jnp.bitwise_and(t + iota, 15)

            tail_cols = [96 + jnp.bitwise_and(d + iota, 3) for d in range(4)]

            @plsc.parallel_loop(0, 96, carry=(neg_inf,) * _K, unroll=4)
            def p1(t, ms):
                x = plsc.load_gather(inbuf, [rows, cols_of(t)])
                tbuf[pl.ds(t * _G, _G)] = x
                return _insert(ms, x)

            ms = p1
            for d in range(4):
                x = plsc.load_gather(inbuf, [rows, tail_cols[d]])
                tbuf[pl.ds((96 + d) * _G, _G)] = x
                ms = _insert(ms, x)
            m0, thr = ms[0], ms[_K - 1]

            @plsc.parallel_loop(0, 96, carry=(zero, zero), unroll=4)
            def p2(t, zs):
                z, s = zs
                x = tbuf[pl.ds(t * _G, _G)]
                e = jnp.where(x >= thr, jnp.exp(x - m0), 0.0)
                mvec = plsc.load_gather(mask_v, [b_vec, cols_of(t)])
                tv = e * mvec
                tbuf[pl.ds(t * _G, _G)] = tv
                return (z + e, s + tv)

            z, s = p2
            for d in range(4):
                x = tbuf[pl.ds((96 + d) * _G, _G)]
                e = jnp.where(x >= thr, jnp.exp(x - m0), 0.0)
                mvec = plsc.load_gather(mask_v, [b_vec, tail_cols[d]])
                tv = e * mvec
                tbuf[pl.ds((96 + d) * _G, _G)] = tv
                z, s = z + e, s + tv
            inv = 1.0 / (s + 1e-8 * z)

            @plsc.parallel_loop(0, 96, unroll=4)
            def p3(t):
                tv = tbuf[pl.ds(t * _G, _G)]
                plsc.store_scatter(outbuf, [rows, cols_of(t)], tv * inv)

            for d in range(4):
                tv = tbuf[pl.ds((96 + d) * _G, _G)]
                plsc.store_scatter(outbuf, [rows, tail_cols[d]], tv * inv)
            return carry2

        lax.fori_loop(0, _GPC, group_body, 0)

    in_copy(0, in0, sin0).start()

    def pair_body(k, carry):
        ci0 = 2 * k
        ci1 = ci0 + 1
        in_copy(ci0, in0, sin0).wait()
        in_copy(ci1, in1, sin1).start()

        @pl.when(k > 0)
        def _():
            out_copy(ci0 - 2, out0, sout0).wait()

        compute(ci0, in0, out0)
        out_copy(ci0, out0, sout0).start()

        in_copy(ci1, in1, sin1).wait()

        @pl.when(k < _NCHUNK // 2 - 1)
        def _():
            in_copy(ci1 + 1, in0, sin0).start()

        @pl.when(k > 0)
        def _():
            out_copy(ci1 - 2, out1, sout1).wait()

        compute(ci1, in1, out1)
        out_copy(ci1, out1, sout1).start()
        return carry

    lax.fori_loop(0, _NCHUNK // 2, pair_body, 0)
    out_copy(_NCHUNK - 2, out0, sout0).wait()
    out_copy(_NCHUNK - 1, out1, sout1).wait()


def kernel(a, mask):
    out = _topk_mask_norm(a.reshape(_R, _C), mask)
    return out.reshape(_B, _CAND, _SW, _C)


# speedup vs baseline: 2.5315x; 1.0270x over previous
import functools

import jax
import jax.numpy as jnp
from jax import lax
from jax.experimental import pallas as pl
from jax.experimental.pallas import tpu as pltpu
from jax.experimental.pallas import tpu_sc as plsc

_B = 64
_CAND = 32
_SW = 64
_C = 100
_K = 10
_R = _B * _CAND * _SW
_NC = 2
_NS = 16
_NW = _NC * _NS
_RPW = _R // _NW
_CHUNK = 128
_NCHUNK = _RPW // _CHUNK
_G = 16
_GPC = _CHUNK // _G
_RPB = _CAND * _SW
_CPB = _RPB // _CHUNK


def _insert(ms, x):
    out = [jnp.maximum(x, ms[0])]
    for i in range(1, len(ms)):
        out.append(jnp.maximum(jnp.minimum(x, ms[i - 1]), ms[i]))
    return tuple(out)


@functools.partial(
    pl.kernel,
    out_type=jax.ShapeDtypeStruct((_R, _C), jnp.float32),
    mesh=plsc.VectorSubcoreMesh(core_axis_name="c", subcore_axis_name="s"),
    compiler_params=pltpu.CompilerParams(needs_layout_passes=False),
    scratch_types=[
        pltpu.VMEM((_CHUNK, _C), jnp.float32),
        pltpu.VMEM((_CHUNK, _C), jnp.float32),
        pltpu.VMEM((_CHUNK, _C), jnp.float32),
        pltpu.VMEM((_CHUNK, _C), jnp.float32),
        pltpu.VMEM((_G * _C,), jnp.float32),
        pltpu.VMEM((2, _C), jnp.float32),
        pltpu.SemaphoreType.DMA,
        pltpu.SemaphoreType.DMA,
        pltpu.SemaphoreType.DMA,
        pltpu.SemaphoreType.DMA,
    ],
)
def _topk_mask_norm(a_hbm, mask_hbm, out_hbm, in0, in1, out0, out1, tbuf,
                    mask_v, sin0, sin1, sout0, sout1):
    wid = lax.axis_index("s") * _NC + lax.axis_index("c")
    row0 = wid * _RPW
    pltpu.sync_copy(mask_hbm.at[pl.ds(2 * wid, 2)], mask_v)
    iota = lax.iota(jnp.int32, 16)
    neg_inf = jnp.full((16,), -jnp.inf, jnp.float32)
    zero = jnp.zeros((16,), jnp.float32)

    def in_copy(ci, buf, sem):
        crow = row0 + ci * _CHUNK
        return pltpu.make_async_copy(a_hbm.at[pl.ds(crow, _CHUNK)], buf, sem)

    def out_copy(ci, buf, sem):
        crow = row0 + ci * _CHUNK
        return pltpu.make_async_copy(buf, out_hbm.at[pl.ds(crow, _CHUNK)], sem)

    def compute(ci, inbuf, outbuf):
        b_local = ci // _CPB
        b_vec = jnp.broadcast_to(jnp.bitwise_and(b_local, 7), (_G,))

        def group_body(g, carry2):
            rows = g * _G + iota

            def cols_of(t):
                return jnp.bitwise_and(t, -16) + jnp.bitwise_and(t + iota, 15)

            tail_cols = [96 + jnp.bitwise_and(d + iota, 3) for d in range(4)]

            @plsc.parallel_loop(0, 96, carry=(neg_inf,) * _K, unroll=8)
            def p1(t, ms):
                x = plsc.load_gather(inbuf, [rows, cols_of(t)])
                tbuf[pl.ds(t * _G, _G)] = x
                return _insert(ms, x)

            ms = p1
            for d in range(4):
                x = plsc.load_gather(inbuf, [rows, tail_cols[d]])
                tbuf[pl.ds((96 + d) * _G, _G)] = x
                ms = _insert(ms, x)
            m0, thr = ms[0], ms[_K - 1]

            @plsc.parallel_loop(0, 96, carry=(zero, zero), unroll=8)
            def p2(t, zs):
                z, s = zs
                x = tbuf[pl.ds(t * _G, _G)]
                e = jnp.where(x >= thr, jnp.exp(x - m0), 0.0)
                mvec = plsc.load_gather(mask_v, [b_vec, cols_of(t)])
                tv = e * mvec
                tbuf[pl.ds(t * _G, _G)] = tv
                return (z + e, s + tv)

            z, s = p2
            for d in range(4):
                x = tbuf[pl.ds((96 + d) * _G, _G)]
                e = jnp.where(x >= thr, jnp.exp(x - m0), 0.0)
                mvec = plsc.load_gather(mask_v, [b_vec, tail_cols[d]])
                tv = e * mvec
                tbuf[pl.ds((96 + d) * _G, _G)] = tv
                z, s = z + e, s + tv
            inv = 1.0 / (s + 1e-8 * z)

            @plsc.parallel_loop(0, 96, unroll=8)
            def p3(t):
                tv = tbuf[pl.ds(t * _G, _G)]
                plsc.store_scatter(outbuf, [rows, cols_of(t)], tv * inv)

            for d in range(4):
                tv = tbuf[pl.ds((96 + d) * _G, _G)]
                plsc.store_scatter(outbuf, [rows, tail_cols[d]], tv * inv)
            return carry2

        lax.fori_loop(0, _GPC, group_body, 0)

    in_copy(0, in0, sin0).start()

    def pair_body(k, carry):
        ci0 = 2 * k
        ci1 = ci0 + 1
        in_copy(ci0, in0, sin0).wait()
        in_copy(ci1, in1, sin1).start()

        @pl.when(k > 0)
        def _():
            out_copy(ci0 - 2, out0, sout0).wait()

        compute(ci0, in0, out0)
        out_copy(ci0, out0, sout0).start()

        in_copy(ci1, in1, sin1).wait()

        @pl.when(k < _NCHUNK // 2 - 1)
        def _():
            in_copy(ci1 + 1, in0, sin0).start()

        @pl.when(k > 0)
        def _():
            out_copy(ci1 - 2, out1, sout1).wait()

        compute(ci1, in1, out1)
        out_copy(ci1, out1, sout1).start()
        return carry

    lax.fori_loop(0, _NCHUNK // 2, pair_body, 0)
    out_copy(_NCHUNK - 2, out0, sout0).wait()
    out_copy(_NCHUNK - 1, out1, sout1).wait()


def kernel(a, mask):
    out = _topk_mask_norm(a.reshape(_R, _C), mask)
    return out.reshape(_B, _CAND, _SW, _C)


# speedup vs baseline: 2.5671x; 1.0141x over previous
import functools

import jax
import jax.numpy as jnp
from jax import lax
from jax.experimental import pallas as pl
from jax.experimental.pallas import tpu as pltpu
from jax.experimental.pallas import tpu_sc as plsc

_B = 64
_CAND = 32
_SW = 64
_C = 100
_K = 10
_R = _B * _CAND * _SW
_NC = 2
_NS = 16
_NW = _NC * _NS
_RPW = _R // _NW
_CHUNK = 128
_NCHUNK = _RPW // _CHUNK
_G = 16
_GPC = _CHUNK // _G
_RPB = _CAND * _SW
_CPB = _RPB // _CHUNK


def _insert(ms, x):
    out = [jnp.maximum(x, ms[0])]
    for i in range(1, len(ms)):
        out.append(jnp.maximum(jnp.minimum(x, ms[i - 1]), ms[i]))
    return tuple(out)


@functools.partial(
    pl.kernel,
    out_type=jax.ShapeDtypeStruct((_R, _C), jnp.float32),
    mesh=plsc.VectorSubcoreMesh(core_axis_name="c", subcore_axis_name="s"),
    compiler_params=pltpu.CompilerParams(needs_layout_passes=False),
    scratch_types=[
        pltpu.VMEM((_CHUNK, _C), jnp.float32),
        pltpu.VMEM((_CHUNK, _C), jnp.float32),
        pltpu.VMEM((_CHUNK, _C), jnp.float32),
        pltpu.VMEM((_CHUNK, _C), jnp.float32),
        pltpu.VMEM((_G * _C,), jnp.float32),
        pltpu.VMEM((_G * _C,), jnp.int32),
        pltpu.VMEM((2, _C), jnp.float32),
        pltpu.SemaphoreType.DMA,
        pltpu.SemaphoreType.DMA,
        pltpu.SemaphoreType.DMA,
        pltpu.SemaphoreType.DMA,
    ],
)
def _topk_mask_norm(a_hbm, mask_hbm, out_hbm, in0, in1, out0, out1, tbuf,
                    idxtab, mask_v, sin0, sin1, sout0, sout1):
    wid = lax.axis_index("s") * _NC + lax.axis_index("c")
    row0 = wid * _RPW
    pltpu.sync_copy(mask_hbm.at[pl.ds(2 * wid, 2)], mask_v)
    iota = lax.iota(jnp.int32, 16)
    neg_inf = jnp.full((16,), -jnp.inf, jnp.float32)
    zero = jnp.zeros((16,), jnp.float32)

    @plsc.parallel_loop(0, 96)
    def _build(t):
        idxtab[pl.ds(t * _G, _G)] = (
            jnp.bitwise_and(t, -16) + jnp.bitwise_and(t + iota, 15))

    for d in range(4):
        idxtab[pl.ds((96 + d) * _G, _G)] = 96 + jnp.bitwise_and(d + iota, 3)

    def in_copy(ci, buf, sem):
        crow = row0 + ci * _CHUNK
        return pltpu.make_async_copy(a_hbm.at[pl.ds(crow, _CHUNK)], buf, sem)

    def out_copy(ci, buf, sem):
        crow = row0 + ci * _CHUNK
        return pltpu.make_async_copy(buf, out_hbm.at[pl.ds(crow, _CHUNK)], sem)

    def compute(ci, inbuf, outbuf):
        b_local = ci // _CPB
        b_vec = jnp.broadcast_to(jnp.bitwise_and(b_local, 7), (_G,))

        def group_body(g, carry2):
            rows = g * _G + iota

            @plsc.parallel_loop(0, _C, carry=(neg_inf,) * _K, unroll=4)
            def p1(t, ms):
                cv = idxtab[pl.ds(t * _G, _G)]
                x = plsc.load_gather(inbuf, [rows, cv])
                tbuf[pl.ds(t * _G, _G)] = x
                return _insert(ms, x)

            ms = p1
            m0, thr = ms[0], ms[_K - 1]

            @plsc.parallel_loop(0, _C, carry=(zero, zero), unroll=4)
            def p2(t, zs):
                z, s = zs
                x = tbuf[pl.ds(t * _G, _G)]
                e = jnp.where(x >= thr, jnp.exp(x - m0), 0.0)
                cv = idxtab[pl.ds(t * _G, _G)]
                mvec = plsc.load_gather(mask_v, [b_vec, cv])
                tv = e * mvec
                tbuf[pl.ds(t * _G, _G)] = tv
                return (z + e, s + tv)

            z, s = p2
            inv = 1.0 / (s + 1e-8 * z)

            @plsc.parallel_loop(0, _C, unroll=4)
            def p3(t):
                tv = tbuf[pl.ds(t * _G, _G)]
                cv = idxtab[pl.ds(t * _G, _G)]
                plsc.store_scatter(outbuf, [rows, cv], tv * inv)

            return carry2

        lax.fori_loop(0, _GPC, group_body, 0)

    in_copy(0, in0, sin0).start()

    def pair_body(k, carry):
        ci0 = 2 * k
        ci1 = ci0 + 1
        in_copy(ci0, in0, sin0).wait()
        in_copy(ci1, in1, sin1).start()

        @pl.when(k > 0)
        def _():
            out_copy(ci0 - 2, out0, sout0).wait()

        compute(ci0, in0, out0)
        out_copy(ci0, out0, sout0).start()

        in_copy(ci1, in1, sin1).wait()

        @pl.when(k < _NCHUNK // 2 - 1)
        def _():
            in_copy(ci1 + 1, in0, sin0).start()

        @pl.when(k > 0)
        def _():
            out_copy(ci1 - 2, out1, sout1).wait()

        compute(ci1, in1, out1)
        out_copy(ci1, out1, sout1).start()
        return carry

    lax.fori_loop(0, _NCHUNK // 2, pair_body, 0)
    out_copy(_NCHUNK - 2, out0, sout0).wait()
    out_copy(_NCHUNK - 1, out1, sout1).wait()


def kernel(a, mask):
    out = _topk_mask_norm(a.reshape(_R, _C), mask)
    return out.reshape(_B, _CAND, _SW, _C)
